# baseline (speedup 1.0000x reference)
import functools

import jax
import jax.numpy as jnp
from jax import lax
from jax.experimental import pallas as pl
from jax.experimental.pallas import tpu as pltpu
from jax.experimental.pallas import tpu_sc as plsc

N = 10000
E = 160000
DIN = 256
H = 4
D = 64
HD = H * D
HHALF = HD // 2
NC = 2
NS = 16
L = 16
ROWW = HHALF + L
EPT = E // NS
CH = 16
NCHUNK = EPT // CH
NEG_SLOPE = 0.2


def _tc_project(x, Wcat, bcat):
    ROWB = 1000

    def body(x_ref, w_ref, b_ref, o_ref):
        acc = jnp.dot(x_ref[...], w_ref[0], preferred_element_type=jnp.float32)
        o_ref[...] = (acc + b_ref[0])[None]

    return pl.pallas_call(
        body,
        grid=(4, N // ROWB),
        in_specs=[
            pl.BlockSpec((ROWB, DIN), lambda j, i: (i, 0)),
            pl.BlockSpec((1, DIN, HHALF), lambda j, i: (j, 0, 0)),
            pl.BlockSpec((1, 1, HHALF), lambda j, i: (j, 0, 0)),
        ],
        out_specs=pl.BlockSpec((1, ROWB, HHALF), lambda j, i: (j, i, 0)),
        out_shape=jax.ShapeDtypeStruct((4, N, HHALF), jnp.float32),
    )(x, Wcat, bcat)


def _sc_edge(table, src_ids, dst_ids, attn_flat):
    mesh = plsc.VectorSubcoreMesh(core_axis_name="c", subcore_axis_name="s")

    @functools.partial(
        pl.kernel,
        out_type=jax.ShapeDtypeStruct((N, HD), jnp.float32),
        mesh=mesh,
        compiler_params=pltpu.CompilerParams(
            needs_layout_passes=False, use_tc_tiling_on_sc=False),
        scratch_types=[
            pltpu.VMEM((EPT,), jnp.int32),
            pltpu.VMEM((EPT,), jnp.int32),
            pltpu.VMEM((2, CH, HHALF), jnp.float32),
            pltpu.VMEM((2, CH, HHALF), jnp.float32),
            pltpu.VMEM((2, CH, ROWW), jnp.float32),
            pltpu.VMEM((L, ROWW), jnp.float32),
            pltpu.VMEM((HHALF,), jnp.float32),
            pltpu.VMEM((8, ROWW), jnp.float32),
            pltpu.VMEM((8, HHALF), jnp.float32),
            pltpu.VMEM_SHARED((N, ROWW), jnp.float32),
            pltpu.SemaphoreType.DMA,
            pltpu.SemaphoreType.DMA,
            pltpu.SemaphoreType.DMA,
            pltpu.SemaphoreType.DMA,
        ],
    )
    def k(table_hbm, src_hbm, dst_hbm, attn_hbm, out_hbm,
          srcv, dstv, fsb, fdb, outb, zbuf, attnv, p2in, p2out,
          acc, gsem0, gsem1, ssem0, ssem1):
        cid = lax.axis_index("c")
        sid = lax.axis_index("s")
        lane = lax.broadcasted_iota(jnp.int32, (L,), 0)
        gsems = (gsem0, gsem1)
        ssems = (ssem0, ssem1)

        zv = jnp.zeros((L,), jnp.float32)
        for r in range(L):
            for kk in range(ROWW // L):
                zbuf[r, pl.ds(kk * L, L)] = zv

        def zbody(t, carry):
            c = sid + NS * t

            @pl.when(c < N // L)
            def _():
                pltpu.sync_copy(zbuf, acc.at[pl.ds(c * L, L)])

            return carry

        lax.fori_loop(0, (N // L + NS - 1) // NS, zbody, 0)

        ebase = sid * EPT
        pltpu.sync_copy(src_hbm.at[pl.ds(ebase, EPT)], srcv)
        pltpu.sync_copy(dst_hbm.at[pl.ds(ebase, EPT)], dstv)
        pltpu.sync_copy(attn_hbm.at[pl.ds(cid * HHALF, HHALF)], attnv)
        av = [attnv[pl.ds(kk * L, L)] for kk in range(HHALF // L)]

        plsc.subcore_barrier()

        fs_off = cid * N
        fd_off = cid * N + 2 * N
        dummy16 = jnp.zeros((CH,), jnp.int32)

        def fire_gathers(j, par):
            e0 = j * CH
            s16 = srcv[pl.ds(e0, CH)] + fs_off
            d16 = dstv[pl.ds(e0, CH)] + fd_off
            pltpu.async_copy(table_hbm.at[s16], fsb.at[par], gsems[par])
            pltpu.async_copy(table_hbm.at[d16], fdb.at[par], gsems[par])

        def wait_gathers(par):
            pltpu.make_async_copy(table_hbm.at[dummy16], fsb.at[par],
                                  gsems[par]).wait()
            pltpu.make_async_copy(table_hbm.at[dummy16], fdb.at[par],
                                  gsems[par]).wait()

        def fire_scatter(j, par):
            d16 = dstv[pl.ds(j * CH, CH)]
            pltpu.async_copy(outb.at[par], acc.at[d16], ssems[par], add=True)

        def wait_scatter(par):
            pltpu.make_async_copy(outb.at[par], acc.at[dummy16],
                                  ssems[par]).wait()

        perm = [lane ^ s for s in (1, 2, 4, 8)]

        def xsum(v):
            for p in perm:
                v = v + jnp.take_along_axis(v, p, axis=0)
            return v

        def compute(par):
            def group(e0):
                lw = jnp.zeros((L,), jnp.float32)
                for ee in range(8):
                    e = e0 + ee
                    t = []
                    for kk in range(8):
                        z = (fsb[par, e, pl.ds(kk * L, L)]
                             + fdb[par, e, pl.ds(kk * L, L)])
                        m = jnp.maximum(z, NEG_SLOPE * z)
                        t.append(m * av[kk])
                    l0v = xsum((t[0] + t[1]) + (t[2] + t[3]))
                    l1v = xsum((t[4] + t[5]) + (t[6] + t[7]))
                    lw = jnp.where(lane == ee, l0v, lw)
                    lw = jnp.where(lane == 8 + ee, l1v, lw)
                ev = jnp.exp(lw)
                for ee in range(8):
                    e = e0 + ee
                    w0v = jnp.take_along_axis(
                        ev, jnp.full((L,), ee, jnp.int32), axis=0)
                    w1v = jnp.take_along_axis(
                        ev, jnp.full((L,), 8 + ee, jnp.int32), axis=0)
                    for kk in range(8):
                        outb[par, e, pl.ds(kk * L, L)] = (
                            fsb[par, e, pl.ds(kk * L, L)]
                            * (w0v if kk < 4 else w1v))
                    dv = jnp.where(lane == 0, w0v,
                                   jnp.where(lane == 1, w1v,
                                             jnp.zeros_like(w0v)))
                    outb[par, e, pl.ds(HHALF, L)] = dv

            @plsc.parallel_loop(0, CH, 8)
            def _(e0):
                group(e0)

        fire_gathers(0, 0)
        fire_gathers(1, 1)
        wait_gathers(0)
        compute(0)
        fire_scatter(0, 0)
        fire_gathers(2, 0)
        wait_gathers(1)
        compute(1)
        fire_scatter(1, 1)

        def pair_body(i, carry):
            j = 2 + 2 * i
            fire_gathers(j + 1, 1)
            wait_gathers(0)
            wait_scatter(0)
            compute(0)
            fire_scatter(j, 0)
            fire_gathers(j + 2, 0)
            wait_gathers(1)
            wait_scatter(1)
            compute(1)
            fire_scatter(j + 1, 1)
            return carry

        lax.fori_loop(0, (NCHUNK - 3) // 2, pair_body, 0)
        wait_gathers(0)
        wait_scatter(0)
        compute(0)
        fire_scatter(NCHUNK - 1, 0)
        wait_scatter(1)
        wait_scatter(0)

        plsc.subcore_barrier()

        def div_body(t, carry):
            c = sid + NS * t

            @pl.when(c < N // 8)
            def _():
                r0 = c * 8
                pltpu.sync_copy(acc.at[pl.ds(r0, 8)], p2in)
                for r in range(8):
                    dvec = p2in[r, pl.ds(HHALF, L)]
                    inv = jnp.where(dvec > 0.0, 1.0 / dvec,
                                    jnp.zeros_like(dvec))
                    i0 = inv[0]
                    i1 = inv[1]
                    for kk in range(8):
                        scal = i0 if kk < 4 else i1
                        p2out[r, pl.ds(kk * L, L)] = (
                            p2in[r, pl.ds(kk * L, L)] * scal)
                pltpu.sync_copy(
                    p2out,
                    out_hbm.at[pl.ds(r0, 8), pl.ds(cid * HHALF, HHALF)])

            return carry

        lax.fori_loop(0, (N // 8 + NS - 1) // NS, div_body, 0)

    return k(table, src_ids, dst_ids, attn_flat)


def kernel(x, edge_index, W_src, b_src, W_dst, b_dst, attn):
    WsT = W_src.T
    WdT = W_dst.T
    Wcat = jnp.stack([WsT[:, :HHALF], WsT[:, HHALF:],
                      WdT[:, :HHALF], WdT[:, HHALF:]])
    bcat = jnp.stack([b_src[:HHALF], b_src[HHALF:],
                      b_dst[:HHALF], b_dst[HHALF:]]).reshape(4, 1, HHALF)
    table = _tc_project(x, Wcat, bcat).reshape(4 * N, HHALF)
    out = _sc_edge(table, edge_index[0], edge_index[1], attn.reshape(HD))
    return out.reshape(N, H, D)


# speedup vs baseline: 1.7605x; 1.7605x over previous
import functools

import jax
import jax.numpy as jnp
from jax import lax
from jax.experimental import pallas as pl
from jax.experimental.pallas import tpu as pltpu
from jax.experimental.pallas import tpu_sc as plsc

N = 10000
E = 160000
DIN = 256
H = 4
D = 64
HD = H * D
HHALF = HD // 2
NC = 2
NS = 16
L = 16
ROWW = HHALF + L
EPT = E // NS
CH = 16
NCHUNK = EPT // CH
NBUF = 3
P2R = 4
NEG_SLOPE = 0.2


def _tc_project(x, Wcat, bcat):
    ROWB = 1000

    def body(x_ref, w_ref, b_ref, o_ref):
        acc = jnp.dot(x_ref[...], w_ref[0], preferred_element_type=jnp.float32)
        o_ref[...] = (acc + b_ref[0])[None]

    return pl.pallas_call(
        body,
        grid=(4, N // ROWB),
        in_specs=[
            pl.BlockSpec((ROWB, DIN), lambda j, i: (i, 0)),
            pl.BlockSpec((1, DIN, HHALF), lambda j, i: (j, 0, 0)),
            pl.BlockSpec((1, 1, HHALF), lambda j, i: (j, 0, 0)),
        ],
        out_specs=pl.BlockSpec((1, ROWB, HHALF), lambda j, i: (j, i, 0)),
        out_shape=jax.ShapeDtypeStruct((4, N, HHALF), jnp.float32),
    )(x, Wcat, bcat)


def _sc_edge(table, src_ids, dst_ids, attn_flat):
    mesh = plsc.VectorSubcoreMesh(core_axis_name="c", subcore_axis_name="s")

    @functools.partial(
        pl.kernel,
        out_type=jax.ShapeDtypeStruct((N, HD), jnp.float32),
        mesh=mesh,
        compiler_params=pltpu.CompilerParams(
            needs_layout_passes=False, use_tc_tiling_on_sc=False),
        scratch_types=[
            pltpu.VMEM((EPT,), jnp.int32),
            pltpu.VMEM((EPT,), jnp.int32),
            pltpu.VMEM((NBUF, CH, HHALF), jnp.float32),
            pltpu.VMEM((NBUF, CH, HHALF), jnp.float32),
            pltpu.VMEM((NBUF, CH, ROWW), jnp.float32),
            pltpu.VMEM((HHALF,), jnp.float32),
            pltpu.VMEM((P2R, ROWW), jnp.float32),
            pltpu.VMEM((P2R, HHALF), jnp.float32),
            pltpu.VMEM_SHARED((N, ROWW), jnp.float32),
            pltpu.SemaphoreType.DMA,
            pltpu.SemaphoreType.DMA,
        ],
    )
    def k(table_hbm, src_hbm, dst_hbm, attn_hbm, out_hbm,
          srcv, dstv, fsb, fdb, outb, attnv, p2in, p2out,
          acc, gsem, ssem):
        cid = lax.axis_index("c")
        sid = lax.axis_index("s")
        lane = lax.broadcasted_iota(jnp.int32, (L,), 0)

        zv = jnp.zeros((L,), jnp.float32)
        for r in range(CH):
            for kk in range(ROWW // L):
                outb[0, r, pl.ds(kk * L, L)] = zv

        def zbody(t, carry):
            c = sid + NS * t

            @pl.when(c < N // L)
            def _():
                pltpu.sync_copy(outb.at[0], acc.at[pl.ds(c * L, L)])

            return carry

        lax.fori_loop(0, (N // L + NS - 1) // NS, zbody, 0)

        ebase = sid * EPT
        pltpu.sync_copy(src_hbm.at[pl.ds(ebase, EPT)], srcv)
        pltpu.sync_copy(dst_hbm.at[pl.ds(ebase, EPT)], dstv)
        pltpu.sync_copy(attn_hbm.at[pl.ds(cid * HHALF, HHALF)], attnv)
        av = [attnv[pl.ds(kk * L, L)] for kk in range(HHALF // L)]

        plsc.subcore_barrier()

        fs_off = cid * N
        fd_off = cid * N + 2 * N
        dummy16 = jnp.zeros((CH,), jnp.int32)

        def fire_gathers(j, par):
            e0 = j * CH
            s16 = srcv[pl.ds(e0, CH)] + fs_off
            d16 = dstv[pl.ds(e0, CH)] + fd_off
            pltpu.async_copy(table_hbm.at[s16], fsb.at[par], gsem)
            pltpu.async_copy(table_hbm.at[d16], fdb.at[par], gsem)

        def wait_gathers():
            pltpu.make_async_copy(table_hbm.at[dummy16], fsb.at[0],
                                  gsem).wait()
            pltpu.make_async_copy(table_hbm.at[dummy16], fdb.at[0],
                                  gsem).wait()

        def fire_scatter(j, par):
            d16 = dstv[pl.ds(j * CH, CH)]
            pltpu.async_copy(outb.at[par], acc.at[d16], ssem, add=True)

        def wait_scatter():
            pltpu.make_async_copy(outb.at[0], acc.at[dummy16], ssem).wait()

        def compute(par):
            def one_edge(e):
                fs = [fsb[par, e, pl.ds(kk * L, L)] for kk in range(8)]
                t = []
                for kk in range(8):
                    z = fs[kk] + fdb[par, e, pl.ds(kk * L, L)]
                    m = jnp.maximum(z, NEG_SLOPE * z)
                    t.append(m * av[kk])
                l0 = jnp.sum((t[0] + t[1]) + (t[2] + t[3]))
                l1 = jnp.sum((t[4] + t[5]) + (t[6] + t[7]))
                w0v = jnp.exp(jnp.full((L,), l0, jnp.float32))
                w1v = jnp.exp(jnp.full((L,), l1, jnp.float32))
                for kk in range(8):
                    outb[par, e, pl.ds(kk * L, L)] = fs[kk] * (
                        w0v if kk < 4 else w1v)
                dv = jnp.where(lane == 0, w0v,
                               jnp.where(lane == 1, w1v,
                                         jnp.zeros_like(w0v)))
                outb[par, e, pl.ds(HHALF, L)] = dv

            @plsc.parallel_loop(0, CH, 1, unroll=8)
            def _(e):
                one_edge(e)

        for kpre in range(NBUF - 1):
            fire_gathers(kpre, kpre)

        def chunk_body(j, carry):
            par = lax.rem(j, NBUF)

            @pl.when(j + NBUF - 1 < NCHUNK)
            def _():
                fire_gathers(j + NBUF - 1, lax.rem(j + NBUF - 1, NBUF))

            wait_gathers()

            @pl.when(j >= NBUF)
            def _():
                wait_scatter()

            compute(par)
            fire_scatter(j, par)
            return carry

        lax.fori_loop(0, NCHUNK, chunk_body, 0)
        for _ in range(NBUF):
            wait_scatter()

        plsc.subcore_barrier()

        def div_body(t, carry):
            c = sid + NS * t

            @pl.when(c < N // P2R)
            def _():
                r0 = c * P2R
                pltpu.sync_copy(acc.at[pl.ds(r0, P2R)], p2in)
                for r in range(P2R):
                    dvec = p2in[r, pl.ds(HHALF, L)]
                    inv = jnp.where(dvec > 0.0, 1.0 / dvec,
                                    jnp.zeros_like(dvec))
                    i0 = inv[0]
                    i1 = inv[1]
                    for kk in range(8):
                        scal = i0 if kk < 4 else i1
                        p2out[r, pl.ds(kk * L, L)] = (
                            p2in[r, pl.ds(kk * L, L)] * scal)
                pltpu.sync_copy(
                    p2out,
                    out_hbm.at[pl.ds(r0, P2R), pl.ds(cid * HHALF, HHALF)])

            return carry

        lax.fori_loop(0, (N // P2R + NS - 1) // NS, div_body, 0)

    return k(table, src_ids, dst_ids, attn_flat)


def kernel(x, edge_index, W_src, b_src, W_dst, b_dst, attn):
    WsT = W_src.T
    WdT = W_dst.T
    Wcat = jnp.stack([WsT[:, :HHALF], WsT[:, HHALF:],
                      WdT[:, :HHALF], WdT[:, HHALF:]])
    bcat = jnp.stack([b_src[:HHALF], b_src[HHALF:],
                      b_dst[:HHALF], b_dst[HHALF:]]).reshape(4, 1, HHALF)
    table = _tc_project(x, Wcat, bcat).reshape(4 * N, HHALF)
    out = _sc_edge(table, edge_index[0], edge_index[1], attn.reshape(HD))
    return out.reshape(N, H, D)


# speedup vs baseline: 1.8263x; 1.0374x over previous
import functools

import jax
import jax.numpy as jnp
from jax import lax
from jax.experimental import pallas as pl
from jax.experimental.pallas import tpu as pltpu
from jax.experimental.pallas import tpu_sc as plsc

N = 10000
E = 160000
DIN = 256
H = 4
D = 64
HD = H * D
HHALF = HD // 2
NC = 2
NS = 16
L = 16
ROWW = HHALF + L
EPT = E // NS
CH = 16
NCHUNK = EPT // CH
NBUF = 3
P2R = 8
NEG_SLOPE = 0.2


def _tc_project(x, Wcat, bcat):
    ROWB = 1000

    def body(x_ref, w_ref, b_ref, o_ref):
        acc = jnp.dot(x_ref[...], w_ref[0], preferred_element_type=jnp.float32)
        o_ref[...] = (acc + b_ref[0])[None]

    return pl.pallas_call(
        body,
        grid=(4, N // ROWB),
        in_specs=[
            pl.BlockSpec((ROWB, DIN), lambda j, i: (i, 0)),
            pl.BlockSpec((1, DIN, HHALF), lambda j, i: (j, 0, 0)),
            pl.BlockSpec((1, 1, HHALF), lambda j, i: (j, 0, 0)),
        ],
        out_specs=pl.BlockSpec((1, ROWB, HHALF), lambda j, i: (j, i, 0)),
        out_shape=jax.ShapeDtypeStruct((4, N, HHALF), jnp.float32),
    )(x, Wcat, bcat)


def _sc_edge(table, src_ids, dst_ids, attn_flat):
    mesh = plsc.VectorSubcoreMesh(core_axis_name="c", subcore_axis_name="s")

    @functools.partial(
        pl.kernel,
        out_type=jax.ShapeDtypeStruct((N, HD), jnp.float32),
        mesh=mesh,
        compiler_params=pltpu.CompilerParams(
            needs_layout_passes=False, use_tc_tiling_on_sc=False),
        scratch_types=[
            pltpu.VMEM((EPT,), jnp.int32),
            pltpu.VMEM((EPT,), jnp.int32),
            pltpu.VMEM((NBUF, CH, HHALF), jnp.float32),
            pltpu.VMEM((NBUF, CH, HHALF), jnp.float32),
            pltpu.VMEM((NBUF, CH, ROWW), jnp.float32),
            pltpu.VMEM((HHALF,), jnp.float32),
            pltpu.VMEM_SHARED((N, ROWW), jnp.float32),
            pltpu.SemaphoreType.DMA,
            pltpu.SemaphoreType.DMA,
        ],
    )
    def k(table_hbm, src_hbm, dst_hbm, attn_hbm, out_hbm,
          srcv, dstv, fsb, fdb, outb, attnv,
          acc, gsem, ssem):
        cid = lax.axis_index("c")
        sid = lax.axis_index("s")
        lane = lax.broadcasted_iota(jnp.int32, (L,), 0)

        zv = jnp.zeros((L,), jnp.float32)
        for r in range(CH):
            for kk in range(ROWW // L):
                outb[0, r, pl.ds(kk * L, L)] = zv

        def zbody(t, carry):
            c = sid + NS * t

            @pl.when(c < N // L)
            def _():
                pltpu.sync_copy(outb.at[0], acc.at[pl.ds(c * L, L)])

            return carry

        lax.fori_loop(0, (N // L + NS - 1) // NS, zbody, 0)

        ebase = sid * EPT
        pltpu.sync_copy(src_hbm.at[pl.ds(ebase, EPT)], srcv)
        pltpu.sync_copy(dst_hbm.at[pl.ds(ebase, EPT)], dstv)
        pltpu.sync_copy(attn_hbm.at[pl.ds(cid * HHALF, HHALF)], attnv)
        av = [attnv[pl.ds(kk * L, L)] for kk in range(HHALF // L)]

        plsc.subcore_barrier()

        fs_off = cid * N
        fd_off = cid * N + 2 * N
        dummy16 = jnp.zeros((CH,), jnp.int32)

        def fire_gathers(j, par):
            e0 = j * CH
            s16 = srcv[pl.ds(e0, CH)] + fs_off
            d16 = dstv[pl.ds(e0, CH)] + fd_off
            pltpu.async_copy(table_hbm.at[s16], fsb.at[par], gsem)
            pltpu.async_copy(table_hbm.at[d16], fdb.at[par], gsem)

        def wait_gathers():
            pltpu.make_async_copy(table_hbm.at[dummy16], fsb.at[0],
                                  gsem).wait()
            pltpu.make_async_copy(table_hbm.at[dummy16], fdb.at[0],
                                  gsem).wait()

        def fire_scatter(j, par):
            d16 = dstv[pl.ds(j * CH, CH)]
            pltpu.async_copy(outb.at[par], acc.at[d16], ssem, add=True)

        def wait_scatter():
            pltpu.make_async_copy(outb.at[0], acc.at[dummy16], ssem).wait()

        def compute(par):
            def one_edge(e):
                fs = [fsb[par, e, pl.ds(kk * L, L)] for kk in range(8)]
                t = []
                for kk in range(8):
                    z = fs[kk] + fdb[par, e, pl.ds(kk * L, L)]
                    m = jnp.maximum(z, NEG_SLOPE * z)
                    t.append(m * av[kk])
                l0 = jnp.sum((t[0] + t[1]) + (t[2] + t[3]))
                l1 = jnp.sum((t[4] + t[5]) + (t[6] + t[7]))
                w0v = jnp.exp(jnp.full((L,), l0, jnp.float32))
                w1v = jnp.exp(jnp.full((L,), l1, jnp.float32))
                for kk in range(8):
                    outb[par, e, pl.ds(kk * L, L)] = fs[kk] * (
                        w0v if kk < 4 else w1v)
                dv = jnp.where(lane == 0, w0v,
                               jnp.where(lane == 1, w1v,
                                         jnp.zeros_like(w0v)))
                outb[par, e, pl.ds(HHALF, L)] = dv

            @plsc.parallel_loop(0, CH, 1, unroll=8)
            def _(e):
                one_edge(e)

        for kpre in range(NBUF - 1):
            fire_gathers(kpre, kpre)

        def chunk_body(j, carry):
            par = lax.rem(j, NBUF)

            @pl.when(j + NBUF - 1 < NCHUNK)
            def _():
                fire_gathers(j + NBUF - 1, lax.rem(j + NBUF - 1, NBUF))

            wait_gathers()

            @pl.when(j >= NBUF)
            def _():
                wait_scatter()

            compute(par)
            fire_scatter(j, par)
            return carry

        lax.fori_loop(0, NCHUNK, chunk_body, 0)
        for _ in range(NBUF):
            wait_scatter()

        plsc.subcore_barrier()

        def div_body(t, carry):
            c = sid + NS * t

            @pl.when(c < N // P2R)
            def _():
                r0 = c * P2R
                pltpu.sync_copy(acc.at[pl.ds(r0, P2R)],
                                outb.at[0, pl.ds(0, P2R)])
                for r in range(P2R):
                    dvec = outb[0, r, pl.ds(HHALF, L)]
                    inv = jnp.where(dvec > 0.0, 1.0 / dvec,
                                    jnp.zeros_like(dvec))
                    i0 = inv[0]
                    i1 = inv[1]
                    for kk in range(8):
                        scal = i0 if kk < 4 else i1
                        fsb[0, r, pl.ds(kk * L, L)] = (
                            outb[0, r, pl.ds(kk * L, L)] * scal)
                pltpu.sync_copy(
                    fsb.at[0, pl.ds(0, P2R)],
                    out_hbm.at[pl.ds(r0, P2R), pl.ds(cid * HHALF, HHALF)])

            return carry

        lax.fori_loop(0, (N // P2R + NS - 1) // NS, div_body, 0)

    return k(table, src_ids, dst_ids, attn_flat)


def kernel(x, edge_index, W_src, b_src, W_dst, b_dst, attn):
    WsT = W_src.T
    WdT = W_dst.T
    Wcat = jnp.stack([WsT[:, :HHALF], WsT[:, HHALF:],
                      WdT[:, :HHALF], WdT[:, HHALF:]])
    bcat = jnp.stack([b_src[:HHALF], b_src[HHALF:],
                      b_dst[:HHALF], b_dst[HHALF:]]).reshape(4, 1, HHALF)
    table = _tc_project(x, Wcat, bcat).reshape(4 * N, HHALF)
    out = _sc_edge(table, edge_index[0], edge_index[1], attn.reshape(HD))
    return out.reshape(N, H, D)
